# baseline (speedup 1.0000x reference)
import functools

import jax
import jax.numpy as jnp
from jax import lax
from jax.experimental import pallas as pl
from jax.experimental.pallas import tpu as pltpu
from jax.experimental.pallas import tpu_sc as plsc

DIM = 256
MAXX = 7
MAXY = 7
NUM_X = 2 * MAXX + 1
N_NODES = 10000
N_EDGES = 160000

NC = 2
NS = 16
NW = NC * NS
L = 16


_BR = 1000
_T = N_NODES // _BR


def _mlp_bn_body(pos_ref, W1_ref, b1_ref, W2_ref, b2_ref, gamma_ref, beta_ref,
                 h_ref, acc_ref):
    p = pl.program_id(0)
    t = pl.program_id(1)

    u1 = jnp.dot(pos_ref[...], W1_ref[...], preferred_element_type=jnp.float32)
    u1 = u1 + b1_ref[...]
    u1 = u1 * jax.nn.sigmoid(u1)
    u = jnp.dot(u1, W2_ref[...], preferred_element_type=jnp.float32)
    u = u + b2_ref[...]

    @pl.when((p == 0) & (t == 0))
    def _():
        acc_ref[...] = jnp.zeros_like(acc_ref)

    @pl.when(p == 0)
    def _():
        acc_ref[0:1, :] += jnp.sum(u, axis=0, keepdims=True)
        acc_ref[1:2, :] += jnp.sum(u * u, axis=0, keepdims=True)

    @pl.when(p == 1)
    def _():
        mean = acc_ref[0:1, :] * (1.0 / N_NODES)
        var = acc_ref[1:2, :] * (1.0 / N_NODES) - mean * mean
        scale = gamma_ref[...] * lax.rsqrt(var + 1e-5)
        shift = beta_ref[...] - mean * scale
        h_ref[...] = u * scale + shift


def _mlp_bn(pos, W1, b1, W2, b2, gamma, beta):
    return pl.pallas_call(
        _mlp_bn_body,
        grid=(2, _T),
        in_specs=[
            pl.BlockSpec((_BR, 6), lambda p, t: (t, 0)),
            pl.BlockSpec((6, 4 * DIM), lambda p, t: (0, 0)),
            pl.BlockSpec((1, 4 * DIM), lambda p, t: (0, 0)),
            pl.BlockSpec((4 * DIM, DIM), lambda p, t: (0, 0)),
            pl.BlockSpec((1, DIM), lambda p, t: (0, 0)),
            pl.BlockSpec((1, DIM), lambda p, t: (0, 0)),
            pl.BlockSpec((1, DIM), lambda p, t: (0, 0)),
        ],
        out_specs=pl.BlockSpec((_BR, DIM), lambda p, t: (t, 0)),
        out_shape=jax.ShapeDtypeStruct((N_NODES, DIM), jnp.float32),
        scratch_shapes=[pltpu.VMEM((2, DIM), jnp.float32)],
        compiler_params=pltpu.CompilerParams(
            dimension_semantics=("arbitrary", "arbitrary")),
    )(pos, W1, b1, W2, b2, gamma, beta)


_CX = 80
_NXCH = N_NODES // _CX
_CE = 128
_NECH = N_EDGES // _CE


def _sc_gather_body(h_hbm, x_hbm, brick_hbm, ea_hbm, etable_hbm,
                    xout_hbm, eout_hbm,
                    xidx_v, eidx_v, ea_v, rows_v, hbuf_v, sem):
    wid = lax.axis_index("s") * NC + lax.axis_index("c")

    etrips = jnp.where(wid < _NECH % NW, _NECH // NW + 1, _NECH // NW)

    iota2 = lax.broadcasted_iota(jnp.int32, (L,), 0) * 2

    def echunk(c, carry):
        base = (wid + NW * c) * _CE
        pltpu.sync_copy(ea_hbm.at[pl.ds(2 * base, 2 * _CE)], ea_v)

        def mkidx(k, carry2):
            a_at = iota2 + k * (2 * L)
            a = plsc.load_gather(ea_v, [a_at])
            b = plsc.load_gather(ea_v, [a_at + 1])
            eidx_v[pl.ds(k * L, L)] = a * NUM_X + b + (MAXX * NUM_X + MAXY)
            return carry2

        lax.fori_loop(0, _CE // L, mkidx, 0)
        pltpu.async_copy(etable_hbm.at[eidx_v], rows_v, sem).wait()
        pltpu.sync_copy(rows_v, eout_hbm.at[pl.ds(base, _CE)])
        return carry

    lax.fori_loop(0, etrips, echunk, 0)

    xtrips = jnp.where(wid < _NXCH % NW, _NXCH // NW + 1, _NXCH // NW)

    def xchunk(c, carry):
        base = (wid + NW * c) * _CX
        pltpu.sync_copy(x_hbm.at[pl.ds(base, _CX)], xidx_v)
        pltpu.async_copy(brick_hbm.at[xidx_v], rows_v.at[pl.ds(0, _CX)],
                         sem).wait()
        pltpu.sync_copy(h_hbm.at[pl.ds(base, _CX)], hbuf_v)

        def addrow(i, carry2):
            for j in range(DIM // L):
                rows_v[i, pl.ds(j * L, L)] = (
                    rows_v[i, pl.ds(j * L, L)] + hbuf_v[i, pl.ds(j * L, L)])
            return carry2

        lax.fori_loop(0, _CX, addrow, 0)
        pltpu.sync_copy(rows_v.at[pl.ds(0, _CX)], xout_hbm.at[pl.ds(base, _CX)])
        return carry

    lax.fori_loop(0, xtrips, xchunk, 0)


def _sc_gather(h, x, brick_table, edge_attr, edge_table):
    mesh = plsc.VectorSubcoreMesh(core_axis_name="c", subcore_axis_name="s",
                                  num_cores=NC, num_subcores=NS)
    f = functools.partial(
        pl.kernel,
        out_type=(jax.ShapeDtypeStruct((N_NODES, DIM), jnp.float32),
                  jax.ShapeDtypeStruct((N_EDGES, DIM), jnp.float32)),
        mesh=mesh,
        compiler_params=pltpu.CompilerParams(needs_layout_passes=False),
        scratch_types=[
            pltpu.VMEM((_CX,), jnp.int32),
            pltpu.VMEM((_CE,), jnp.int32),
            pltpu.VMEM((2 * _CE,), jnp.int32),
            pltpu.VMEM((_CE, DIM), jnp.float32),
            pltpu.VMEM((_CX, DIM), jnp.float32),
            pltpu.SemaphoreType.DMA,
        ],
    )(_sc_gather_body)
    return f(h, x, brick_table, edge_attr, edge_table)


def kernel(x, pos, edge_attr, brick_table, W1, b1, W2, b2, gamma, beta,
           edge_table):
    h = _mlp_bn(pos, W1, b1.reshape(1, -1), W2, b2.reshape(1, -1),
                gamma.reshape(1, -1), beta.reshape(1, -1))
    x_emb, e_emb = _sc_gather(h, x.astype(jnp.int32), brick_table,
                              edge_attr.astype(jnp.int32).reshape(-1),
                              edge_table)
    return (x_emb, e_emb)
